# Initial kernel scaffold; baseline (speedup 1.0000x reference)
import jax
import jax.numpy as jnp
from jax.experimental import pallas as pl


def kernel(x, edge_index, W1, a_src1, a_dst1, b1, W2, a_src2, a_dst2, b2, Wc, bc):
    raise NotImplementedError("write your pallas kernel here")


# speedup vs baseline: 1.1016x; 1.1016x over previous
import functools

import jax
import jax.numpy as jnp
from jax import lax
from jax.experimental import pallas as pl
from jax.experimental.pallas import tpu as pltpu

N = 10000
D = 256
NP = 10240
BLK = 512
GRID = NP // BLK
WPAD = 144
HALF = 128

_INTERPRET = False


def _dense_body(x_blk, W, a_s, a_d):
    h = jnp.dot(x_blk, W, preferred_element_type=jnp.float32)
    asv = jnp.dot(h, a_s, preferred_element_type=jnp.float32)
    adv = jnp.dot(h, a_d, preferred_element_type=jnp.float32)
    return h, asv, adv


def _pack_half(h_half):
    ones = jnp.ones((h_half.shape[0], 1), jnp.float32)
    zeros = jnp.zeros((h_half.shape[0], WPAD - HALF - 1), jnp.float32)
    return jnp.concatenate([h_half, ones, zeros], axis=1)


def _layer1_kernel(x_ref, W_ref, as_ref, ad_ref,
                   h0_ref, h1_ref, asv_ref, adv_ref, mx_ref):
    i = pl.program_id(0)
    h, asv, adv = _dense_body(x_ref[...], W_ref[...], as_ref[...], ad_ref[...])
    h0_ref[...] = _pack_half(h[:, :HALF])
    h1_ref[...] = _pack_half(h[:, HALF:])
    asv_ref[...] = asv[None, None, :]
    adv_ref[...] = adv[None, None, :]

    @pl.when(i == 0)
    def _():
        mx_ref[...] = jnp.full((1, 2), -jnp.inf, jnp.float32)

    cur = mx_ref[...]
    new = jnp.stack([jnp.max(asv), jnp.max(adv)])[None, :]
    mx_ref[...] = jnp.maximum(cur, new)


def _layer_next_kernel(acc0_ref, acc1_ref, b_ref, W_ref, as_ref, ad_ref,
                       h0_ref, h1_ref, asv_ref, adv_ref, mx_ref):
    i = pl.program_id(0)
    a0 = acc0_ref[...]
    a1 = acc1_ref[...]
    den = jnp.maximum(a0[:, HALF:HALF + 1], 1e-30)
    x = jnp.concatenate([a0[:, :HALF], a1[:, :HALF]], axis=1) / den
    x = jnp.maximum(x + b_ref[...][None, :], 0.0)
    h, asv, adv = _dense_body(x, W_ref[...], as_ref[...], ad_ref[...])
    h0_ref[...] = _pack_half(h[:, :HALF])
    h1_ref[...] = _pack_half(h[:, HALF:])
    asv_ref[...] = asv[None, None, :]
    adv_ref[...] = adv[None, None, :]

    @pl.when(i == 0)
    def _():
        mx_ref[...] = jnp.full((1, 2), -jnp.inf, jnp.float32)

    cur = mx_ref[...]
    new = jnp.stack([jnp.max(asv), jnp.max(adv)])[None, :]
    mx_ref[...] = jnp.maximum(cur, new)


def _final_kernel(acc0_ref, acc1_ref, b_ref, Wc_ref, bc_ref, out_ref):
    a0 = acc0_ref[...]
    a1 = acc1_ref[...]
    den = jnp.maximum(a0[:, HALF:HALF + 1], 1e-30)
    x = jnp.concatenate([a0[:, :HALF], a1[:, :HALF]], axis=1) / den
    x = jnp.maximum(x + b_ref[...][None, :], 0.0)
    logits = jnp.dot(x, Wc_ref[...], preferred_element_type=jnp.float32)
    logits = logits + bc_ref[...][None, :]
    m = jnp.max(logits, axis=1, keepdims=True)
    lse = m + jnp.log(jnp.sum(jnp.exp(logits - m), axis=1, keepdims=True))
    out_ref[...] = logits - lse


def _row_spec(width):
    return pl.BlockSpec((BLK, width), lambda i: (i, 0))


def _full_spec(shape):
    nd = len(shape)
    return pl.BlockSpec(shape, lambda i: (0,) * nd)


def _vec_spec():
    return pl.BlockSpec((1, 1, BLK), lambda i: (i, 0, 0))


def _dense_layer1(x_p, W, a_s, a_d):
    return pl.pallas_call(
        _layer1_kernel,
        grid=(GRID,),
        in_specs=[_row_spec(D), _full_spec((D, D)), _full_spec((D,)),
                  _full_spec((D,))],
        out_specs=[_row_spec(WPAD), _row_spec(WPAD), _vec_spec(), _vec_spec(),
                   pl.BlockSpec((1, 2), lambda i: (0, 0))],
        out_shape=[
            jax.ShapeDtypeStruct((NP, WPAD), jnp.float32),
            jax.ShapeDtypeStruct((NP, WPAD), jnp.float32),
            jax.ShapeDtypeStruct((GRID, 1, BLK), jnp.float32),
            jax.ShapeDtypeStruct((GRID, 1, BLK), jnp.float32),
            jax.ShapeDtypeStruct((1, 2), jnp.float32),
        ],
        interpret=_INTERPRET,
    )(x_p, W, a_s, a_d)


def _dense_next(acc0, acc1, b, W, a_s, a_d):
    return pl.pallas_call(
        _layer_next_kernel,
        grid=(GRID,),
        in_specs=[_row_spec(WPAD), _row_spec(WPAD), _full_spec((D,)),
                  _full_spec((D, D)), _full_spec((D,)), _full_spec((D,))],
        out_specs=[_row_spec(WPAD), _row_spec(WPAD), _vec_spec(), _vec_spec(),
                   pl.BlockSpec((1, 2), lambda i: (0, 0))],
        out_shape=[
            jax.ShapeDtypeStruct((NP, WPAD), jnp.float32),
            jax.ShapeDtypeStruct((NP, WPAD), jnp.float32),
            jax.ShapeDtypeStruct((GRID, 1, BLK), jnp.float32),
            jax.ShapeDtypeStruct((GRID, 1, BLK), jnp.float32),
            jax.ShapeDtypeStruct((1, 2), jnp.float32),
        ],
        interpret=_INTERPRET,
    )(acc0, acc1, b, W, a_s, a_d)


def _dense_final(acc0, acc1, b, Wc, bc):
    C = Wc.shape[1]
    return pl.pallas_call(
        _final_kernel,
        grid=(GRID,),
        in_specs=[_row_spec(WPAD), _row_spec(WPAD), _full_spec((D,)),
                  _full_spec((D, C)), _full_spec((C,))],
        out_specs=pl.BlockSpec((BLK, C), lambda i: (i, 0)),
        out_shape=jax.ShapeDtypeStruct((NP, C), jnp.float32),
        interpret=_INTERPRET,
    )(acc0, acc1, b, Wc, bc)


def _edge_phase_jnp(h0, h1, asv, adv, mx, src, dst):
    ms = mx[0, 0] + mx[0, 1]
    bound = jnp.where(ms > 0, ms, 0.2 * ms)
    s = asv[src] + adv[dst]
    e = jnp.where(s > 0, s, 0.2 * s)
    w = jnp.exp(e - bound)
    acc0 = jax.ops.segment_sum(h0[src] * w[:, None], dst, num_segments=NP)
    acc1 = jax.ops.segment_sum(h1[src] * w[:, None], dst, num_segments=NP)
    return acc0, acc1


def kernel(x, edge_index, W1, a_src1, a_dst1, b1, W2, a_src2, a_dst2, b2,
           Wc, bc):
    x_p = jnp.pad(x, ((0, NP - N), (0, 0)))
    loop = jnp.arange(N, dtype=edge_index.dtype)
    src = jnp.concatenate([edge_index[0], loop])
    dst = jnp.concatenate([edge_index[1], loop])

    h0, h1, asv2d, adv2d, mx = _dense_layer1(x_p, W1, a_src1, a_dst1)
    asv, adv = asv2d.reshape(-1), adv2d.reshape(-1)
    acc0, acc1 = _edge_phase_jnp(h0, h1, asv, adv, mx, src, dst)

    h0, h1, asv2d, adv2d, mx = _dense_next(acc0, acc1, b1, W2, a_src2, a_dst2)
    asv, adv = asv2d.reshape(-1), adv2d.reshape(-1)
    acc0, acc1 = _edge_phase_jnp(h0, h1, asv, adv, mx, src, dst)

    out = _dense_final(acc0, acc1, b2, Wc, bc)
    return out[:N]


# speedup vs baseline: 11.2480x; 10.2106x over previous
import functools

import jax
import jax.numpy as jnp
from jax import lax
from jax.experimental import pallas as pl
from jax.experimental.pallas import tpu as pltpu
from jax.experimental.pallas import tpu_sc as plsc

N = 10000
D = 256
NP = 10240
BLK = 512
GRID = NP // BLK
WPAD = 144
HALF = 128

NS = 16
NC = 2
K = 112
E_TOT = 160000 + N
CHUNKS = -(-E_TOT // (NS * K))
E_PAD = NS * K * CHUNKS
PT = E_PAD // NS
ROWS_PT = NP // NS
RCOPY = 80

_INTERPRET = False


def _dense_body(x_blk, W, a_s, a_d):
    h = jnp.dot(x_blk, W, preferred_element_type=jnp.float32)
    asv = jnp.dot(h, a_s, preferred_element_type=jnp.float32)
    adv = jnp.dot(h, a_d, preferred_element_type=jnp.float32)
    return h, asv, adv


def _pack_half(h_half):
    ones = jnp.ones((h_half.shape[0], 1), jnp.float32)
    zeros = jnp.zeros((h_half.shape[0], WPAD - HALF - 1), jnp.float32)
    return jnp.concatenate([h_half, ones, zeros], axis=1)


def _layer1_kernel(x_ref, W_ref, as_ref, ad_ref,
                   h0_ref, h1_ref, asv_ref, adv_ref, mx_ref):
    i = pl.program_id(0)
    h, asv, adv = _dense_body(x_ref[...], W_ref[...], as_ref[...], ad_ref[...])
    h0_ref[...] = _pack_half(h[:, :HALF])
    h1_ref[...] = _pack_half(h[:, HALF:])
    asv_ref[...] = asv[None, None, :]
    adv_ref[...] = adv[None, None, :]

    @pl.when(i == 0)
    def _():
        mx_ref[...] = jnp.full((1, 2), -jnp.inf, jnp.float32)

    cur = mx_ref[...]
    new = jnp.stack([jnp.max(asv), jnp.max(adv)])[None, :]
    mx_ref[...] = jnp.maximum(cur, new)


def _layer_next_kernel(acc0_ref, acc1_ref, b_ref, W_ref, as_ref, ad_ref,
                       h0_ref, h1_ref, asv_ref, adv_ref, mx_ref):
    i = pl.program_id(0)
    a0 = acc0_ref[...]
    a1 = acc1_ref[...]
    den = jnp.maximum(a0[:, HALF:HALF + 1], 1e-30)
    x = jnp.concatenate([a0[:, :HALF], a1[:, :HALF]], axis=1) / den
    x = jnp.maximum(x + b_ref[...][None, :], 0.0)
    h, asv, adv = _dense_body(x, W_ref[...], as_ref[...], ad_ref[...])
    h0_ref[...] = _pack_half(h[:, :HALF])
    h1_ref[...] = _pack_half(h[:, HALF:])
    asv_ref[...] = asv[None, None, :]
    adv_ref[...] = adv[None, None, :]

    @pl.when(i == 0)
    def _():
        mx_ref[...] = jnp.full((1, 2), -jnp.inf, jnp.float32)

    cur = mx_ref[...]
    new = jnp.stack([jnp.max(asv), jnp.max(adv)])[None, :]
    mx_ref[...] = jnp.maximum(cur, new)


def _final_kernel(acc0_ref, acc1_ref, b_ref, Wc_ref, bc_ref, out_ref):
    a0 = acc0_ref[...]
    a1 = acc1_ref[...]
    den = jnp.maximum(a0[:, HALF:HALF + 1], 1e-30)
    x = jnp.concatenate([a0[:, :HALF], a1[:, :HALF]], axis=1) / den
    x = jnp.maximum(x + b_ref[...][None, :], 0.0)
    logits = jnp.dot(x, Wc_ref[...], preferred_element_type=jnp.float32)
    logits = logits + bc_ref[...][None, :]
    m = jnp.max(logits, axis=1, keepdims=True)
    lse = m + jnp.log(jnp.sum(jnp.exp(logits - m), axis=1, keepdims=True))
    out_ref[...] = logits - lse


def _row_spec(width):
    return pl.BlockSpec((BLK, width), lambda i: (i, 0))


def _full_spec(shape):
    nd = len(shape)
    return pl.BlockSpec(shape, lambda i: (0,) * nd)


def _vec_spec():
    return pl.BlockSpec((1, 1, BLK), lambda i: (i, 0, 0))


def _dense_layer1(x_p, W, a_s, a_d):
    return pl.pallas_call(
        _layer1_kernel,
        grid=(GRID,),
        in_specs=[_row_spec(D), _full_spec((D, D)), _full_spec((D,)),
                  _full_spec((D,))],
        out_specs=[_row_spec(WPAD), _row_spec(WPAD), _vec_spec(), _vec_spec(),
                   pl.BlockSpec((1, 2), lambda i: (0, 0))],
        out_shape=[
            jax.ShapeDtypeStruct((NP, WPAD), jnp.float32),
            jax.ShapeDtypeStruct((NP, WPAD), jnp.float32),
            jax.ShapeDtypeStruct((GRID, 1, BLK), jnp.float32),
            jax.ShapeDtypeStruct((GRID, 1, BLK), jnp.float32),
            jax.ShapeDtypeStruct((1, 2), jnp.float32),
        ],
        interpret=_INTERPRET,
    )(x_p, W, a_s, a_d)


def _dense_next(acc0, acc1, b, W, a_s, a_d):
    return pl.pallas_call(
        _layer_next_kernel,
        grid=(GRID,),
        in_specs=[_row_spec(WPAD), _row_spec(WPAD), _full_spec((D,)),
                  _full_spec((D, D)), _full_spec((D,)), _full_spec((D,))],
        out_specs=[_row_spec(WPAD), _row_spec(WPAD), _vec_spec(), _vec_spec(),
                   pl.BlockSpec((1, 2), lambda i: (0, 0))],
        out_shape=[
            jax.ShapeDtypeStruct((NP, WPAD), jnp.float32),
            jax.ShapeDtypeStruct((NP, WPAD), jnp.float32),
            jax.ShapeDtypeStruct((GRID, 1, BLK), jnp.float32),
            jax.ShapeDtypeStruct((GRID, 1, BLK), jnp.float32),
            jax.ShapeDtypeStruct((1, 2), jnp.float32),
        ],
        interpret=_INTERPRET,
    )(acc0, acc1, b, W, a_s, a_d)


def _dense_final(acc0, acc1, b, Wc, bc):
    C = Wc.shape[1]
    return pl.pallas_call(
        _final_kernel,
        grid=(GRID,),
        in_specs=[_row_spec(WPAD), _row_spec(WPAD), _full_spec((D,)),
                  _full_spec((D, C)), _full_spec((C,))],
        out_specs=pl.BlockSpec((BLK, C), lambda i: (i, 0)),
        out_shape=jax.ShapeDtypeStruct((NP, C), jnp.float32),
        interpret=_INTERPRET,
    )(acc0, acc1, b, Wc, bc)


def _sc_edge_body(h0_hbm, h1_hbm, as_hbm, ad_hbm, m_hbm, src_hbm, dst_hbm,
                  acc0_hbm, acc1_hbm,
                  acc_sh, as_v, ad_v, m_v, src_v, dst_v, w_v, rows_v, sem):
    c = lax.axis_index("c")
    s = lax.axis_index("s")

    pltpu.sync_copy(as_hbm, as_v)
    pltpu.sync_copy(ad_hbm, ad_v)
    pltpu.sync_copy(m_hbm, m_v)
    mvec = m_v[...]

    def _zrow(r, carry):
        for j in range(WPAD // 16):
            rows_v[r, pl.ds(j * 16, 16)] = jnp.zeros((16,), jnp.float32)
        return carry

    lax.fori_loop(0, K, _zrow, 0)
    base_r = s * ROWS_PT
    for k in range(ROWS_PT // RCOPY):
        pltpu.sync_copy(rows_v.at[pl.ds(0, RCOPY)],
                        acc_sh.at[pl.ds(base_r + k * RCOPY, RCOPY)])
    plsc.subcore_barrier()

    ebase = s * PT

    def _run(h_hbm, acc_hbm):
        def chunk_body(g, carry):
            off = ebase + g * K
            pltpu.sync_copy(src_hbm.at[pl.ds(off, K)], src_v)
            pltpu.sync_copy(dst_hbm.at[pl.ds(off, K)], dst_v)
            cp = pltpu.async_copy(h_hbm.at[src_v], rows_v, sem)

            def wbody(j, carry2):
                sidx = src_v[pl.ds(j * 16, 16)]
                didx = dst_v[pl.ds(j * 16, 16)]
                sv = plsc.load_gather(as_v, [sidx]) + plsc.load_gather(
                    ad_v, [didx])
                e = jnp.where(sv > 0, sv, 0.2 * sv)
                w_v[pl.ds(j * 16, 16)] = jnp.exp(e - mvec)
                return carry2

            lax.fori_loop(0, K // 16, wbody, 0)
            cp.wait()

            def rbody(r, carry2):
                wr = plsc.load_gather(w_v, [jnp.full((16,), r, jnp.int32)])
                for j in range(WPAD // 16):
                    v = rows_v[r, pl.ds(j * 16, 16)]
                    rows_v[r, pl.ds(j * 16, 16)] = v * wr
                return carry2

            lax.fori_loop(0, K, rbody, 0)
            pltpu.sync_copy(rows_v, acc_sh.at[dst_v], add=True)
            return carry

        lax.fori_loop(0, PT // K, chunk_body, 0)
        plsc.subcore_barrier()
        for k in range(ROWS_PT // RCOPY):
            pltpu.sync_copy(acc_sh.at[pl.ds(base_r + k * RCOPY, RCOPY)],
                            rows_v.at[pl.ds(0, RCOPY)])
            pltpu.sync_copy(rows_v.at[pl.ds(0, RCOPY)],
                            acc_hbm.at[pl.ds(base_r + k * RCOPY, RCOPY)])

    @pl.when(c == 0)
    def _():
        _run(h0_hbm, acc0_hbm)

    @pl.when(c == 1)
    def _():
        _run(h1_hbm, acc1_hbm)


@functools.partial(
    pl.kernel,
    out_type=[
        jax.ShapeDtypeStruct((NP, WPAD), jnp.float32),
        jax.ShapeDtypeStruct((NP, WPAD), jnp.float32),
    ],
    mesh=plsc.VectorSubcoreMesh(core_axis_name="c", subcore_axis_name="s",
                                num_cores=NC, num_subcores=NS),
    compiler_params=pltpu.CompilerParams(needs_layout_passes=False,
                                         use_tc_tiling_on_sc=False),
    scratch_types=[
        pltpu.MemorySpace.VMEM_SHARED((NP, WPAD), jnp.float32),
        pltpu.VMEM((NP,), jnp.float32),
        pltpu.VMEM((NP,), jnp.float32),
        pltpu.VMEM((16,), jnp.float32),
        pltpu.VMEM((K,), jnp.int32),
        pltpu.VMEM((K,), jnp.int32),
        pltpu.VMEM((K,), jnp.float32),
        pltpu.VMEM((K, WPAD), jnp.float32),
        pltpu.SemaphoreType.DMA,
    ],
)
def _sc_edge_kernel(h0_hbm, h1_hbm, as_hbm, ad_hbm, m_hbm, src_hbm, dst_hbm,
                    acc0_hbm, acc1_hbm, acc_sh, as_v, ad_v, m_v, src_v,
                    dst_v, w_v, rows_v, sem):
    _sc_edge_body(h0_hbm, h1_hbm, as_hbm, ad_hbm, m_hbm, src_hbm, dst_hbm,
                  acc0_hbm, acc1_hbm, acc_sh, as_v, ad_v, m_v, src_v, dst_v,
                  w_v, rows_v, sem)


def _edge_phase_jnp(h0, h1, asv, adv, mx, src, dst):
    ms = mx[0, 0] + mx[0, 1]
    bound = jnp.where(ms > 0, ms, 0.2 * ms)
    s = asv[src] + adv[dst]
    e = jnp.where(s > 0, s, 0.2 * s)
    w = jnp.exp(e - bound)
    acc0 = jax.ops.segment_sum(h0[src] * w[:, None], dst, num_segments=NP)
    acc1 = jax.ops.segment_sum(h1[src] * w[:, None], dst, num_segments=NP)
    return acc0, acc1


def _edge_phase_sc(h0, h1, asv, adv, mx, src, dst):
    ms = mx[0, 0] + mx[0, 1]
    bound = jnp.where(ms > 0, ms, 0.2 * ms)
    marr = jnp.full((16,), bound, jnp.float32)
    return _sc_edge_kernel(h0, h1, asv, adv, marr, src, dst)


def kernel(x, edge_index, W1, a_src1, a_dst1, b1, W2, a_src2, a_dst2, b2,
           Wc, bc):
    x_p = jnp.pad(x, ((0, NP - N), (0, 0)))
    loop = jnp.arange(N, dtype=edge_index.dtype)
    pad_idx = jnp.full((E_PAD - E_TOT,), NP - 1, dtype=edge_index.dtype)
    src = jnp.concatenate([edge_index[0], loop, pad_idx])
    dst = jnp.concatenate([edge_index[1], loop, pad_idx])

    h0, h1, asv2d, adv2d, mx = _dense_layer1(x_p, W1, a_src1, a_dst1)
    asv, adv = asv2d.reshape(-1), adv2d.reshape(-1)
    acc0, acc1 = _edge_phase_sc(h0, h1, asv, adv, mx, src, dst)

    h0, h1, asv2d, adv2d, mx = _dense_next(acc0, acc1, b1, W2, a_src2, a_dst2)
    asv, adv = asv2d.reshape(-1), adv2d.reshape(-1)
    acc0, acc1 = _edge_phase_sc(h0, h1, asv, adv, mx, src, dst)

    out = _dense_final(acc0, acc1, b2, Wc, bc)
    return out[:N]


# speedup vs baseline: 11.9101x; 1.0589x over previous
import functools

import jax
import jax.numpy as jnp
from jax import lax
from jax.experimental import pallas as pl
from jax.experimental.pallas import tpu as pltpu
from jax.experimental.pallas import tpu_sc as plsc

N = 10000
D = 256
NP = 10240
BLK = 512
GRID = NP // BLK
WPAD = 144
HALF = 128

NS = 16
NC = 2
K = 128
E_TOT = 160000 + N
E_PAD = 172032
PT = E_PAD // NS
CH = PT // K
KW = 448
PTW = E_PAD // (NS * NC)
CHW = PTW // KW
ROWS_PT = NP // NS
RCOPY = 128

_INTERPRET = False


def _dense_body(x_blk, W, a_s, a_d):
    h = jnp.dot(x_blk, W, preferred_element_type=jnp.float32)
    asv = jnp.dot(h, a_s, preferred_element_type=jnp.float32)
    adv = jnp.dot(h, a_d, preferred_element_type=jnp.float32)
    return h, asv, adv


def _pack_half(h_half):
    ones = jnp.ones((h_half.shape[0], 1), jnp.float32)
    zeros = jnp.zeros((h_half.shape[0], WPAD - HALF - 1), jnp.float32)
    return jnp.concatenate([h_half, ones, zeros], axis=1)


def _layer1_kernel(x_ref, W_ref, as_ref, ad_ref,
                   h0_ref, h1_ref, asv_ref, adv_ref, mx_ref):
    i = pl.program_id(0)
    h, asv, adv = _dense_body(x_ref[...], W_ref[...], as_ref[...], ad_ref[...])
    h0_ref[...] = _pack_half(h[:, :HALF])
    h1_ref[...] = _pack_half(h[:, HALF:])
    asv_ref[...] = asv[None, None, :]
    adv_ref[...] = adv[None, None, :]

    @pl.when(i == 0)
    def _():
        mx_ref[...] = jnp.full((1, 2), -jnp.inf, jnp.float32)

    cur = mx_ref[...]
    new = jnp.stack([jnp.max(asv), jnp.max(adv)])[None, :]
    mx_ref[...] = jnp.maximum(cur, new)


def _layer_next_kernel(acc0_ref, acc1_ref, b_ref, W_ref, as_ref, ad_ref,
                       h0_ref, h1_ref, asv_ref, adv_ref, mx_ref):
    i = pl.program_id(0)
    a0 = acc0_ref[...]
    a1 = acc1_ref[...]
    den = jnp.maximum(a0[:, HALF:HALF + 1], 1e-30)
    x = jnp.concatenate([a0[:, :HALF], a1[:, :HALF]], axis=1) / den
    x = jnp.maximum(x + b_ref[...][None, :], 0.0)
    h, asv, adv = _dense_body(x, W_ref[...], as_ref[...], ad_ref[...])
    h0_ref[...] = _pack_half(h[:, :HALF])
    h1_ref[...] = _pack_half(h[:, HALF:])
    asv_ref[...] = asv[None, None, :]
    adv_ref[...] = adv[None, None, :]

    @pl.when(i == 0)
    def _():
        mx_ref[...] = jnp.full((1, 2), -jnp.inf, jnp.float32)

    cur = mx_ref[...]
    new = jnp.stack([jnp.max(asv), jnp.max(adv)])[None, :]
    mx_ref[...] = jnp.maximum(cur, new)


def _final_kernel(acc0_ref, acc1_ref, b_ref, Wc_ref, bc_ref, out_ref):
    a0 = acc0_ref[...]
    a1 = acc1_ref[...]
    den = jnp.maximum(a0[:, HALF:HALF + 1], 1e-30)
    x = jnp.concatenate([a0[:, :HALF], a1[:, :HALF]], axis=1) / den
    x = jnp.maximum(x + b_ref[...][None, :], 0.0)
    logits = jnp.dot(x, Wc_ref[...], preferred_element_type=jnp.float32)
    logits = logits + bc_ref[...][None, :]
    m = jnp.max(logits, axis=1, keepdims=True)
    lse = m + jnp.log(jnp.sum(jnp.exp(logits - m), axis=1, keepdims=True))
    out_ref[...] = logits - lse


def _row_spec(width):
    return pl.BlockSpec((BLK, width), lambda i: (i, 0))


def _full_spec(shape):
    nd = len(shape)
    return pl.BlockSpec(shape, lambda i: (0,) * nd)


def _vec_spec():
    return pl.BlockSpec((1, 1, BLK), lambda i: (i, 0, 0))


def _dense_layer1(x_p, W, a_s, a_d):
    return pl.pallas_call(
        _layer1_kernel,
        grid=(GRID,),
        in_specs=[_row_spec(D), _full_spec((D, D)), _full_spec((D,)),
                  _full_spec((D,))],
        out_specs=[_row_spec(WPAD), _row_spec(WPAD), _vec_spec(), _vec_spec(),
                   pl.BlockSpec((1, 2), lambda i: (0, 0))],
        out_shape=[
            jax.ShapeDtypeStruct((NP, WPAD), jnp.float32),
            jax.ShapeDtypeStruct((NP, WPAD), jnp.float32),
            jax.ShapeDtypeStruct((GRID, 1, BLK), jnp.float32),
            jax.ShapeDtypeStruct((GRID, 1, BLK), jnp.float32),
            jax.ShapeDtypeStruct((1, 2), jnp.float32),
        ],
        interpret=_INTERPRET,
    )(x_p, W, a_s, a_d)


def _dense_next(acc0, acc1, b, W, a_s, a_d):
    return pl.pallas_call(
        _layer_next_kernel,
        grid=(GRID,),
        in_specs=[_row_spec(WPAD), _row_spec(WPAD), _full_spec((D,)),
                  _full_spec((D, D)), _full_spec((D,)), _full_spec((D,))],
        out_specs=[_row_spec(WPAD), _row_spec(WPAD), _vec_spec(), _vec_spec(),
                   pl.BlockSpec((1, 2), lambda i: (0, 0))],
        out_shape=[
            jax.ShapeDtypeStruct((NP, WPAD), jnp.float32),
            jax.ShapeDtypeStruct((NP, WPAD), jnp.float32),
            jax.ShapeDtypeStruct((GRID, 1, BLK), jnp.float32),
            jax.ShapeDtypeStruct((GRID, 1, BLK), jnp.float32),
            jax.ShapeDtypeStruct((1, 2), jnp.float32),
        ],
        interpret=_INTERPRET,
    )(acc0, acc1, b, W, a_s, a_d)


def _dense_final(acc0, acc1, b, Wc, bc):
    C = Wc.shape[1]
    return pl.pallas_call(
        _final_kernel,
        grid=(GRID,),
        in_specs=[_row_spec(WPAD), _row_spec(WPAD), _full_spec((D,)),
                  _full_spec((D, C)), _full_spec((C,))],
        out_specs=pl.BlockSpec((BLK, C), lambda i: (i, 0)),
        out_shape=jax.ShapeDtypeStruct((NP, C), jnp.float32),
        interpret=_INTERPRET,
    )(acc0, acc1, b, Wc, bc)


@functools.partial(
    pl.kernel,
    out_type=jax.ShapeDtypeStruct((E_PAD,), jnp.float32),
    mesh=plsc.VectorSubcoreMesh(core_axis_name="c", subcore_axis_name="s",
                                num_cores=NC, num_subcores=NS),
    compiler_params=pltpu.CompilerParams(needs_layout_passes=False,
                                         use_tc_tiling_on_sc=False),
    scratch_types=[
        pltpu.VMEM((NP,), jnp.float32),
        pltpu.VMEM((NP,), jnp.float32),
        pltpu.VMEM((16,), jnp.float32),
        pltpu.VMEM((KW,), jnp.int32),
        pltpu.VMEM((KW,), jnp.int32),
        pltpu.VMEM((KW,), jnp.float32),
    ],
)
def _sc_w_kernel(as_hbm, ad_hbm, m_hbm, src_hbm, dst_hbm, w_hbm,
                 as_v, ad_v, m_v, src_v, dst_v, w_v):
    c = lax.axis_index("c")
    s = lax.axis_index("s")
    wid = s * NC + c

    pltpu.sync_copy(as_hbm, as_v)
    pltpu.sync_copy(ad_hbm, ad_v)
    pltpu.sync_copy(m_hbm, m_v)
    mvec = m_v[...]
    base = wid * PTW

    def chunk_body(g, carry):
        off = base + g * KW
        pltpu.sync_copy(src_hbm.at[pl.ds(off, KW)], src_v)
        pltpu.sync_copy(dst_hbm.at[pl.ds(off, KW)], dst_v)

        def wbody(j, carry2):
            sidx = src_v[pl.ds(j * 16, 16)]
            didx = dst_v[pl.ds(j * 16, 16)]
            sv = plsc.load_gather(as_v, [sidx]) + plsc.load_gather(
                ad_v, [didx])
            e = jnp.where(sv > 0, sv, 0.2 * sv)
            w_v[pl.ds(j * 16, 16)] = jnp.exp(e - mvec)
            return carry2

        lax.fori_loop(0, KW // 16, wbody, 0)

        pltpu.sync_copy(w_v, w_hbm.at[pl.ds(off, KW)])
        return carry

    lax.fori_loop(0, CHW, chunk_body, 0)


def _scale_rows(rows_b, w_v):
    def rbody(r, carry):
        wr = plsc.load_gather(w_v, [jnp.full((16,), r, jnp.int32)])
        for j in range(WPAD // 16):
            v = rows_b[r, pl.ds(j * 16, 16)]
            rows_b[r, pl.ds(j * 16, 16)] = v * wr
        return carry

    lax.fori_loop(0, K, rbody, 0)


@functools.partial(
    pl.kernel,
    out_type=[
        jax.ShapeDtypeStruct((NP, WPAD), jnp.float32),
        jax.ShapeDtypeStruct((NP, WPAD), jnp.float32),
    ],
    mesh=plsc.VectorSubcoreMesh(core_axis_name="c", subcore_axis_name="s",
                                num_cores=NC, num_subcores=NS),
    compiler_params=pltpu.CompilerParams(needs_layout_passes=False,
                                         use_tc_tiling_on_sc=False),
    scratch_types=[
        pltpu.MemorySpace.VMEM_SHARED((NP, WPAD), jnp.float32),
        pltpu.VMEM((K,), jnp.int32),
        pltpu.VMEM((K,), jnp.int32),
        pltpu.VMEM((K,), jnp.int32),
        pltpu.VMEM((K,), jnp.int32),
        pltpu.VMEM((K,), jnp.float32),
        pltpu.VMEM((K, WPAD), jnp.float32),
        pltpu.VMEM((K, WPAD), jnp.float32),
        pltpu.SemaphoreType.DMA,
        pltpu.SemaphoreType.DMA,
        pltpu.SemaphoreType.DMA,
        pltpu.SemaphoreType.DMA,
    ],
)
def _sc_scatter_kernel(h0_hbm, h1_hbm, w_hbm, src_hbm, dst_hbm,
                       acc0_hbm, acc1_hbm, acc_sh,
                       src0, dst0, src1, dst1, w_v, rows0, rows1,
                       gsem0, gsem1, ssem0, ssem1):
    c = lax.axis_index("c")
    s = lax.axis_index("s")

    def _zrow(r, carry):
        for j in range(WPAD // 16):
            rows0[r, pl.ds(j * 16, 16)] = jnp.zeros((16,), jnp.float32)
        return carry

    lax.fori_loop(0, K, _zrow, 0)
    base_r = s * ROWS_PT
    for k in range(ROWS_PT // RCOPY):
        pltpu.sync_copy(rows0.at[pl.ds(0, RCOPY)],
                        acc_sh.at[pl.ds(base_r + k * RCOPY, RCOPY)])
    plsc.subcore_barrier()

    ebase = s * PT
    bufs = ((src0, dst0, rows0, gsem0, ssem0),
            (src1, dst1, rows1, gsem1, ssem1))

    def _run(h_hbm, acc_hbm):
        def load_idx_and_gather(g, par):
            src_b, dst_b, rows_b, gsem, _ = bufs[par]
            off = ebase + g * K
            pltpu.sync_copy(src_hbm.at[pl.ds(off, K)], src_b)
            pltpu.sync_copy(dst_hbm.at[pl.ds(off, K)], dst_b)
            pltpu.async_copy(h_hbm.at[src_b], rows_b, gsem)

        def process(g, par):
            src_b, dst_b, rows_b, gsem, ssem = bufs[par]
            pltpu.make_async_copy(h_hbm.at[src_b], rows_b, gsem).wait()
            pltpu.sync_copy(w_hbm.at[pl.ds(ebase + g * K, K)], w_v)
            _scale_rows(rows_b, w_v)
            pltpu.async_copy(rows_b, acc_sh.at[dst_b], ssem, add=True)

        def wait_scatter(par):
            src_b, dst_b, rows_b, _, ssem = bufs[par]
            pltpu.make_async_copy(rows_b, acc_sh.at[dst_b], ssem).wait()

        load_idx_and_gather(0, 0)
        load_idx_and_gather(1, 1)

        def steady(t, carry):
            g0 = 2 * t
            process(g0, 0)
            process(g0 + 1, 1)
            wait_scatter(0)
            load_idx_and_gather(g0 + 2, 0)
            wait_scatter(1)
            load_idx_and_gather(g0 + 3, 1)
            return carry

        lax.fori_loop(0, CH // 2 - 1, steady, 0)

        process(CH - 2, 0)
        process(CH - 1, 1)
        wait_scatter(0)
        wait_scatter(1)

        plsc.subcore_barrier()
        for k in range(ROWS_PT // RCOPY):
            pltpu.sync_copy(acc_sh.at[pl.ds(base_r + k * RCOPY, RCOPY)],
                            rows0)
            pltpu.sync_copy(rows0,
                            acc_hbm.at[pl.ds(base_r + k * RCOPY, RCOPY)])

    @pl.when(c == 0)
    def _():
        _run(h0_hbm, acc0_hbm)

    @pl.when(c == 1)
    def _():
        _run(h1_hbm, acc1_hbm)


def _edge_phase_jnp(h0, h1, asv, adv, mx, src, dst):
    ms = mx[0, 0] + mx[0, 1]
    bound = jnp.where(ms > 0, ms, 0.2 * ms)
    s = asv[src] + adv[dst]
    e = jnp.where(s > 0, s, 0.2 * s)
    w = jnp.exp(e - bound)
    acc0 = jax.ops.segment_sum(h0[src] * w[:, None], dst, num_segments=NP)
    acc1 = jax.ops.segment_sum(h1[src] * w[:, None], dst, num_segments=NP)
    return acc0, acc1


def _edge_phase_sc(h0, h1, asv, adv, mx, src, dst):
    ms = mx[0, 0] + mx[0, 1]
    bound = jnp.where(ms > 0, ms, 0.2 * ms)
    marr = jnp.full((16,), bound, jnp.float32)
    w = _sc_w_kernel(asv, adv, marr, src, dst)
    return _sc_scatter_kernel(h0, h1, w, src, dst)


def kernel(x, edge_index, W1, a_src1, a_dst1, b1, W2, a_src2, a_dst2, b2,
           Wc, bc):
    x_p = jnp.pad(x, ((0, NP - N), (0, 0)))
    loop = jnp.arange(N, dtype=edge_index.dtype)
    pad_idx = jnp.full((E_PAD - E_TOT,), NP - 1, dtype=edge_index.dtype)
    src = jnp.concatenate([edge_index[0], loop, pad_idx])
    dst = jnp.concatenate([edge_index[1], loop, pad_idx])

    h0, h1, asv2d, adv2d, mx = _dense_layer1(x_p, W1, a_src1, a_dst1)
    asv, adv = asv2d.reshape(-1), adv2d.reshape(-1)
    acc0, acc1 = _edge_phase_sc(h0, h1, asv, adv, mx, src, dst)

    h0, h1, asv2d, adv2d, mx = _dense_next(acc0, acc1, b1, W2, a_src2, a_dst2)
    asv, adv = asv2d.reshape(-1), adv2d.reshape(-1)
    acc0, acc1 = _edge_phase_sc(h0, h1, asv, adv, mx, src, dst)

    out = _dense_final(acc0, acc1, b2, Wc, bc)
    return out[:N]


# speedup vs baseline: 13.9334x; 1.1699x over previous
import functools

import jax
import jax.numpy as jnp
from jax import lax
from jax.experimental import pallas as pl
from jax.experimental.pallas import tpu as pltpu
from jax.experimental.pallas import tpu_sc as plsc

N = 10000
D = 256
NP = 10240
BLK = 512
GRID = NP // BLK
WPAD = 144
HALF = 128

NS = 16
NC = 2
K = 112
E_TOT = 160000 + N
E_PAD = 172032
PT = E_PAD // NS
CH = PT // K
SUP = 12
NSUP = CH // SUP
PTW = E_PAD // (NS * NC)
ROWS_PT = NP // NS
RCOPY = 80

_INTERPRET = False


def _dense_body(x_blk, W, a_s, a_d):
    h = jnp.dot(x_blk, W, preferred_element_type=jnp.float32)
    asv = jnp.dot(h, a_s, preferred_element_type=jnp.float32)
    adv = jnp.dot(h, a_d, preferred_element_type=jnp.float32)
    return h, asv, adv


def _pack_half(h_half):
    ones = jnp.ones((h_half.shape[0], 1), jnp.float32)
    zeros = jnp.zeros((h_half.shape[0], WPAD - HALF - 1), jnp.float32)
    return jnp.concatenate([h_half, ones, zeros], axis=1)


def _layer1_kernel(x_ref, W_ref, as_ref, ad_ref,
                   h0_ref, h1_ref, asv_ref, adv_ref, mx_ref):
    i = pl.program_id(0)
    h, asv, adv = _dense_body(x_ref[...], W_ref[...], as_ref[...], ad_ref[...])
    h0_ref[...] = _pack_half(h[:, :HALF])
    h1_ref[...] = _pack_half(h[:, HALF:])
    asv_ref[...] = asv[None, None, :]
    adv_ref[...] = adv[None, None, :]

    @pl.when(i == 0)
    def _():
        mx_ref[...] = jnp.full((1, 2), -jnp.inf, jnp.float32)

    cur = mx_ref[...]
    new = jnp.stack([jnp.max(asv), jnp.max(adv)])[None, :]
    mx_ref[...] = jnp.maximum(cur, new)


def _layer_next_kernel(acc0_ref, acc1_ref, b_ref, W_ref, as_ref, ad_ref,
                       h0_ref, h1_ref, asv_ref, adv_ref, mx_ref):
    i = pl.program_id(0)
    a0 = acc0_ref[...]
    a1 = acc1_ref[...]
    den = jnp.maximum(a0[:, HALF:HALF + 1], 1e-30)
    x = jnp.concatenate([a0[:, :HALF], a1[:, :HALF]], axis=1) / den
    x = jnp.maximum(x + b_ref[...][None, :], 0.0)
    h, asv, adv = _dense_body(x, W_ref[...], as_ref[...], ad_ref[...])
    h0_ref[...] = _pack_half(h[:, :HALF])
    h1_ref[...] = _pack_half(h[:, HALF:])
    asv_ref[...] = asv[None, None, :]
    adv_ref[...] = adv[None, None, :]

    @pl.when(i == 0)
    def _():
        mx_ref[...] = jnp.full((1, 2), -jnp.inf, jnp.float32)

    cur = mx_ref[...]
    new = jnp.stack([jnp.max(asv), jnp.max(adv)])[None, :]
    mx_ref[...] = jnp.maximum(cur, new)


def _final_kernel(acc0_ref, acc1_ref, b_ref, Wc_ref, bc_ref, out_ref):
    a0 = acc0_ref[...]
    a1 = acc1_ref[...]
    den = jnp.maximum(a0[:, HALF:HALF + 1], 1e-30)
    x = jnp.concatenate([a0[:, :HALF], a1[:, :HALF]], axis=1) / den
    x = jnp.maximum(x + b_ref[...][None, :], 0.0)
    logits = jnp.dot(x, Wc_ref[...], preferred_element_type=jnp.float32)
    logits = logits + bc_ref[...][None, :]
    m = jnp.max(logits, axis=1, keepdims=True)
    lse = m + jnp.log(jnp.sum(jnp.exp(logits - m), axis=1, keepdims=True))
    out_ref[...] = logits - lse


def _row_spec(width):
    return pl.BlockSpec((BLK, width), lambda i: (i, 0))


def _full_spec(shape):
    nd = len(shape)
    return pl.BlockSpec(shape, lambda i: (0,) * nd)


def _vec_spec():
    return pl.BlockSpec((1, 1, BLK), lambda i: (i, 0, 0))


def _dense_layer1(x_p, W, a_s, a_d):
    return pl.pallas_call(
        _layer1_kernel,
        grid=(GRID,),
        in_specs=[_row_spec(D), _full_spec((D, D)), _full_spec((D,)),
                  _full_spec((D,))],
        out_specs=[_row_spec(WPAD), _row_spec(WPAD), _vec_spec(), _vec_spec(),
                   pl.BlockSpec((1, 2), lambda i: (0, 0))],
        out_shape=[
            jax.ShapeDtypeStruct((NP, WPAD), jnp.float32),
            jax.ShapeDtypeStruct((NP, WPAD), jnp.float32),
            jax.ShapeDtypeStruct((GRID, 1, BLK), jnp.float32),
            jax.ShapeDtypeStruct((GRID, 1, BLK), jnp.float32),
            jax.ShapeDtypeStruct((1, 2), jnp.float32),
        ],
        interpret=_INTERPRET,
    )(x_p, W, a_s, a_d)


def _dense_next(acc0, acc1, b, W, a_s, a_d):
    return pl.pallas_call(
        _layer_next_kernel,
        grid=(GRID,),
        in_specs=[_row_spec(WPAD), _row_spec(WPAD), _full_spec((D,)),
                  _full_spec((D, D)), _full_spec((D,)), _full_spec((D,))],
        out_specs=[_row_spec(WPAD), _row_spec(WPAD), _vec_spec(), _vec_spec(),
                   pl.BlockSpec((1, 2), lambda i: (0, 0))],
        out_shape=[
            jax.ShapeDtypeStruct((NP, WPAD), jnp.float32),
            jax.ShapeDtypeStruct((NP, WPAD), jnp.float32),
            jax.ShapeDtypeStruct((GRID, 1, BLK), jnp.float32),
            jax.ShapeDtypeStruct((GRID, 1, BLK), jnp.float32),
            jax.ShapeDtypeStruct((1, 2), jnp.float32),
        ],
        interpret=_INTERPRET,
    )(acc0, acc1, b, W, a_s, a_d)


def _dense_final(acc0, acc1, b, Wc, bc):
    C = Wc.shape[1]
    return pl.pallas_call(
        _final_kernel,
        grid=(GRID,),
        in_specs=[_row_spec(WPAD), _row_spec(WPAD), _full_spec((D,)),
                  _full_spec((D, C)), _full_spec((C,))],
        out_specs=pl.BlockSpec((BLK, C), lambda i: (i, 0)),
        out_shape=jax.ShapeDtypeStruct((NP, C), jnp.float32),
        interpret=_INTERPRET,
    )(acc0, acc1, b, Wc, bc)


@functools.partial(
    pl.kernel,
    out_type=jax.ShapeDtypeStruct((E_PAD,), jnp.float32),
    mesh=plsc.VectorSubcoreMesh(core_axis_name="c", subcore_axis_name="s",
                                num_cores=NC, num_subcores=NS),
    compiler_params=pltpu.CompilerParams(needs_layout_passes=False,
                                         use_tc_tiling_on_sc=False),
    scratch_types=[
        pltpu.VMEM((NP,), jnp.float32),
        pltpu.VMEM((NP,), jnp.float32),
        pltpu.VMEM((16,), jnp.float32),
        pltpu.VMEM((PTW,), jnp.int32),
        pltpu.VMEM((PTW,), jnp.int32),
        pltpu.VMEM((PTW,), jnp.float32),
    ],
)
def _sc_w_kernel(as_hbm, ad_hbm, m_hbm, src_hbm, dst_hbm, w_hbm,
                 as_v, ad_v, m_v, src_v, dst_v, w_v):
    c = lax.axis_index("c")
    s = lax.axis_index("s")
    wid = s * NC + c

    pltpu.sync_copy(as_hbm, as_v)
    pltpu.sync_copy(ad_hbm, ad_v)
    pltpu.sync_copy(m_hbm, m_v)
    mvec = m_v[...]
    base = wid * PTW
    pltpu.sync_copy(src_hbm.at[pl.ds(base, PTW)], src_v)
    pltpu.sync_copy(dst_hbm.at[pl.ds(base, PTW)], dst_v)

    def wbody(j, carry):
        for u in range(2):
            o = j * 32 + u * 16
            sidx = src_v[pl.ds(o, 16)]
            didx = dst_v[pl.ds(o, 16)]
            sv = plsc.load_gather(as_v, [sidx]) + plsc.load_gather(
                ad_v, [didx])
            e = jnp.where(sv > 0, sv, 0.2 * sv)
            w_v[pl.ds(o, 16)] = jnp.exp(e - mvec)
        return carry

    lax.fori_loop(0, PTW // 32, wbody, 0)
    pltpu.sync_copy(w_v, w_hbm.at[pl.ds(base, PTW)])


def _scale_rows(rows_b, w_v, wbase, emask):
    def rbody(q, carry):
        r0 = 2 * q
        r1 = r0 + 1
        w0 = plsc.load_gather(w_v, [jnp.full((16,), wbase + r0, jnp.int32)])
        w1 = plsc.load_gather(w_v, [jnp.full((16,), wbase + r1, jnp.int32)])
        for j in range(HALF // 16):
            sl = pl.ds(j * 16, 16)
            rows_b[r0, sl] = rows_b[r0, sl] * w0
            rows_b[r1, sl] = rows_b[r1, sl] * w1
        sl = pl.ds(HALF, 16)
        rows_b[r0, sl] = w0 * emask
        rows_b[r1, sl] = w1 * emask
        return carry

    lax.fori_loop(0, K // 2, rbody, 0)


@functools.partial(
    pl.kernel,
    out_type=[
        jax.ShapeDtypeStruct((NP, WPAD), jnp.float32),
        jax.ShapeDtypeStruct((NP, WPAD), jnp.float32),
    ],
    mesh=plsc.VectorSubcoreMesh(core_axis_name="c", subcore_axis_name="s",
                                num_cores=NC, num_subcores=NS),
    compiler_params=pltpu.CompilerParams(needs_layout_passes=False,
                                         use_tc_tiling_on_sc=False),
    scratch_types=[
        pltpu.MemorySpace.VMEM_SHARED((NP, WPAD), jnp.float32),
        pltpu.VMEM((SUP, K), jnp.int32),
        pltpu.VMEM((SUP, K), jnp.int32),
        pltpu.VMEM((SUP * K,), jnp.float32),
        pltpu.VMEM((K, WPAD), jnp.float32),
        pltpu.VMEM((K, WPAD), jnp.float32),
        pltpu.SemaphoreType.DMA,
        pltpu.SemaphoreType.DMA,
        pltpu.SemaphoreType.DMA,
        pltpu.SemaphoreType.DMA,
    ],
)
def _sc_scatter_kernel(h0_hbm, h1_hbm, w_hbm, src2_hbm, dst2_hbm,
                       acc0_hbm, acc1_hbm, acc_sh,
                       src_sup, dst_sup, w_v, rows0, rows1,
                       gsem0, gsem1, ssem0, ssem1):
    c = lax.axis_index("c")
    s = lax.axis_index("s")
    emask = jnp.where(jnp.arange(16, dtype=jnp.int32) == 0,
                      jnp.float32(1), jnp.float32(0))

    def _zrow(r, carry):
        for j in range(WPAD // 16):
            rows0[r, pl.ds(j * 16, 16)] = jnp.zeros((16,), jnp.float32)
        return carry

    lax.fori_loop(0, K, _zrow, 0)
    base_r = s * ROWS_PT
    for k in range(ROWS_PT // RCOPY):
        pltpu.sync_copy(rows0.at[pl.ds(0, RCOPY)],
                        acc_sh.at[pl.ds(base_r + k * RCOPY, RCOPY)])
    plsc.subcore_barrier()

    bufs = ((rows0, gsem0, ssem0), (rows1, gsem1, ssem1))

    def _run(h_hbm, acc_hbm):
        def superchunk(u, carry):
            crow = s * CH + u * SUP
            pltpu.sync_copy(src2_hbm.at[pl.ds(crow, SUP)], src_sup)
            pltpu.sync_copy(dst2_hbm.at[pl.ds(crow, SUP)], dst_sup)
            pltpu.sync_copy(w_hbm.at[pl.ds(crow * K, SUP * K)], w_v)

            def gather(k):
                rows_b, gsem, _ = bufs[k % 2]
                pltpu.async_copy(h_hbm.at[src_sup.at[k]], rows_b, gsem)

            def process(k):
                rows_b, gsem, ssem = bufs[k % 2]
                pltpu.make_async_copy(h_hbm.at[src_sup.at[k]], rows_b,
                                      gsem).wait()
                _scale_rows(rows_b, w_v, k * K, emask)
                pltpu.async_copy(rows_b, acc_sh.at[dst_sup.at[k]], ssem,
                                 add=True)

            def wait_scatter(k):
                rows_b, _, ssem = bufs[k % 2]
                pltpu.make_async_copy(rows_b, acc_sh.at[dst_sup.at[k]],
                                      ssem).wait()

            gather(0)
            gather(1)
            for t in range(SUP // 2):
                process(2 * t)
                process(2 * t + 1)
                if 2 * t + 2 < SUP:
                    wait_scatter(2 * t)
                    gather(2 * t + 2)
                    wait_scatter(2 * t + 1)
                    gather(2 * t + 3)
            wait_scatter(SUP - 2)
            wait_scatter(SUP - 1)
            return carry

        lax.fori_loop(0, NSUP, superchunk, 0)

        plsc.subcore_barrier()
        for k in range(ROWS_PT // RCOPY):
            pltpu.sync_copy(acc_sh.at[pl.ds(base_r + k * RCOPY, RCOPY)],
                            rows0.at[pl.ds(0, RCOPY)])
            pltpu.sync_copy(rows0.at[pl.ds(0, RCOPY)],
                            acc_hbm.at[pl.ds(base_r + k * RCOPY, RCOPY)])

    @pl.when(c == 0)
    def _():
        _run(h0_hbm, acc0_hbm)

    @pl.when(c == 1)
    def _():
        _run(h1_hbm, acc1_hbm)


def _edge_phase_jnp(h0, h1, asv, adv, mx, src, dst):
    ms = mx[0, 0] + mx[0, 1]
    bound = jnp.where(ms > 0, ms, 0.2 * ms)
    s = asv[src] + adv[dst]
    e = jnp.where(s > 0, s, 0.2 * s)
    w = jnp.exp(e - bound)
    acc0 = jax.ops.segment_sum(h0[src] * w[:, None], dst, num_segments=NP)
    acc1 = jax.ops.segment_sum(h1[src] * w[:, None], dst, num_segments=NP)
    return acc0, acc1


def _edge_phase_sc(h0, h1, asv, adv, mx, src, dst, src2, dst2):
    ms = mx[0, 0] + mx[0, 1]
    bound = jnp.where(ms > 0, ms, 0.2 * ms)
    marr = jnp.full((16,), bound, jnp.float32)
    w = _sc_w_kernel(asv, adv, marr, src, dst)
    return _sc_scatter_kernel(h0, h1, w, src2, dst2)


def kernel(x, edge_index, W1, a_src1, a_dst1, b1, W2, a_src2, a_dst2, b2,
           Wc, bc):
    x_p = jnp.pad(x, ((0, NP - N), (0, 0)))
    loop = jnp.arange(N, dtype=edge_index.dtype)
    pad_idx = jnp.full((E_PAD - E_TOT,), NP - 1, dtype=edge_index.dtype)
    src = jnp.concatenate([edge_index[0], loop, pad_idx])
    dst = jnp.concatenate([edge_index[1], loop, pad_idx])
    src2 = src.reshape(E_PAD // K, K)
    dst2 = dst.reshape(E_PAD // K, K)

    h0, h1, asv2d, adv2d, mx = _dense_layer1(x_p, W1, a_src1, a_dst1)
    asv, adv = asv2d.reshape(-1), adv2d.reshape(-1)
    acc0, acc1 = _edge_phase_sc(h0, h1, asv, adv, mx, src, dst, src2, dst2)

    h0, h1, asv2d, adv2d, mx = _dense_next(acc0, acc1, b1, W2, a_src2, a_dst2)
    asv, adv = asv2d.reshape(-1), adv2d.reshape(-1)
    acc0, acc1 = _edge_phase_sc(h0, h1, asv, adv, mx, src, dst, src2, dst2)

    out = _dense_final(acc0, acc1, b2, Wc, bc)
    return out[:N]
